# baseline (speedup 1.0000x reference)
import functools

import jax
import jax.numpy as jnp
from jax import lax
from jax.experimental import pallas as pl
from jax.experimental.pallas import tpu as pltpu
from jax.experimental.pallas import tpu_sc as plsc

BATCH = 16384
FIELDS = 26
D = 64
N_ROWS = BATCH * FIELDS

NC = 2
NS = 16
NW = NC * NS
PER_W = N_ROWS // NW
CH = 128
G = 4
GROUP = CH * G
NCH = PER_W // CH
NG = PER_W // GROUP

_sc_mesh = plsc.VectorSubcoreMesh(
    core_axis_name="c", subcore_axis_name="s", num_cores=NC, num_subcores=NS
)


@functools.partial(
    pl.kernel,
    mesh=_sc_mesh,
    out_type=jax.ShapeDtypeStruct((N_ROWS, D), jnp.float32),
    scratch_types=[
        pltpu.VMEM((NCH, CH), jnp.int32),
        pltpu.VMEM((GROUP, D), jnp.float32),
        pltpu.SemaphoreType.DMA,
    ],
    compiler_params=pltpu.CompilerParams(use_tc_tiling_on_sc=False),
)
def _sc_gather(idx_hbm, table_hbm, out_hbm, idx_v, rows_v, gsem):
    wid = lax.axis_index("s") * NC + lax.axis_index("c")
    base = wid * PER_W
    pltpu.sync_copy(idx_hbm.at[wid], idx_v)

    def group_body(g, carry):
        copies = [
            pltpu.async_copy(
                table_hbm.at[idx_v.at[g * G + j]],
                rows_v.at[pl.ds(j * CH, CH)],
                gsem,
            )
            for j in range(G)
        ]
        for c in copies:
            c.wait()
        pltpu.sync_copy(rows_v, out_hbm.at[pl.ds(base + g * GROUP, GROUP)])
        return carry

    lax.fori_loop(0, NG, group_body, 0)


_SQRT_HALF = 0.7071067811865476


def _mlp_body(x_ref, w1_ref, b1_ref, w2_ref, b2_ref, o_ref):
    x = x_ref[...]
    h = jnp.dot(x, w1_ref[...], preferred_element_type=jnp.float32) + b1_ref[...]
    h = h * 0.5 * (1.0 + lax.erf(h * _SQRT_HALF))
    o_ref[...] = (
        jnp.dot(h, w2_ref[...], preferred_element_type=jnp.float32) + b2_ref[...]
    )


BLK = 2048


def _mlp(embed, w1t, b1, w2t, b2):
    return pl.pallas_call(
        _mlp_body,
        grid=(N_ROWS // BLK,),
        in_specs=[
            pl.BlockSpec((BLK, D), lambda i: (i, 0)),
            pl.BlockSpec((D, D), lambda i: (0, 0)),
            pl.BlockSpec((1, D), lambda i: (0, 0)),
            pl.BlockSpec((D, D), lambda i: (0, 0)),
            pl.BlockSpec((1, D), lambda i: (0, 0)),
        ],
        out_specs=pl.BlockSpec((BLK, D), lambda i: (i, 0)),
        out_shape=jax.ShapeDtypeStruct((N_ROWS, D), jnp.float32),
    )(embed, w1t, b1, w2t, b2)


def kernel(action_ids, W_emb, W1, b1, W2, b2):
    idx = action_ids.reshape(NW, NCH, CH)
    embed = _sc_gather(idx, W_emb)
    out = _mlp(embed, W1.T, b1.reshape(1, D), W2.T, b2.reshape(1, D))
    return out.reshape(BATCH, FIELDS, D)


# speedup vs baseline: 1.7106x; 1.7106x over previous
import functools

import jax
import jax.numpy as jnp
from jax import lax
from jax.experimental import pallas as pl
from jax.experimental.pallas import tpu as pltpu
from jax.experimental.pallas import tpu_sc as plsc

BATCH = 16384
FIELDS = 26
D = 64
N_ROWS = BATCH * FIELDS
HALF_B = BATCH // 2

NC = 2
NS = 16
NW = NC * NS
PER_W = N_ROWS // NW
CH = 128
G = 4
GROUP = CH * G
NCH = PER_W // CH
NG = PER_W // GROUP

_sc_mesh = plsc.VectorSubcoreMesh(
    core_axis_name="c", subcore_axis_name="s", num_cores=NC, num_subcores=NS
)


@functools.partial(
    pl.kernel,
    mesh=_sc_mesh,
    out_type=jax.ShapeDtypeStruct((N_ROWS, D), jnp.float32),
    scratch_types=[
        pltpu.VMEM((NCH, CH), jnp.int32),
        pltpu.VMEM((GROUP, D), jnp.float32),
        pltpu.SemaphoreType.DMA,
    ],
    compiler_params=pltpu.CompilerParams(use_tc_tiling_on_sc=False),
)
def _sc_gather(idx_hbm, table_hbm, out_hbm, idx_v, rows_v, gsem):
    wid = lax.axis_index("s") * NC + lax.axis_index("c")
    base = wid * PER_W
    pltpu.sync_copy(idx_hbm.at[wid], idx_v)

    def group_body(g, carry):
        copies = [
            pltpu.async_copy(
                table_hbm.at[idx_v.at[g * G + j]],
                rows_v.at[pl.ds(j * CH, CH)],
                gsem,
            )
            for j in range(G)
        ]
        for c in copies:
            c.wait()
        pltpu.sync_copy(rows_v, out_hbm.at[pl.ds(base + g * GROUP, GROUP)])
        return carry

    lax.fori_loop(0, NG, group_body, 0)


_SQRT_HALF = 0.7071067811865476


def _mlp_body(x_ref, w1l_ref, w1r_ref, b1_ref, w2_ref, b2c_ref, o_ref):
    x = x_ref[...]
    b1 = b1_ref[...]
    b2c = b2c_ref[...]
    w2 = w2_ref[...]
    for half, w1_ref in ((0, w1l_ref), (1, w1r_ref)):
        h = jnp.dot(x, w1_ref[...], preferred_element_type=jnp.float32) + b1
        h = h * 0.5 * (1.0 + lax.erf(h * _SQRT_HALF))
        y = lax.dot_general(
            w2, h, (((1,), (1,)), ((), ())), preferred_element_type=jnp.float32
        )
        o_ref[0, :, pl.ds(half * HALF_B, HALF_B)] = y + b2c


def _mlp(staging128, w1l, w1r, b1, w2, b2c):
    return pl.pallas_call(
        _mlp_body,
        grid=(FIELDS,),
        in_specs=[
            pl.BlockSpec((HALF_B, 2 * D), lambda f: (f, 0)),
            pl.BlockSpec((2 * D, D), lambda f: (0, 0)),
            pl.BlockSpec((2 * D, D), lambda f: (0, 0)),
            pl.BlockSpec((1, D), lambda f: (0, 0)),
            pl.BlockSpec((D, D), lambda f: (0, 0)),
            pl.BlockSpec((D, 1), lambda f: (0, 0)),
        ],
        out_specs=pl.BlockSpec((1, D, BATCH), lambda f: (f, 0, 0)),
        out_shape=jax.ShapeDtypeStruct((FIELDS, D, BATCH), jnp.float32),
    )(staging128, w1l, w1r, b1, w2, b2c)


def kernel(action_ids, W_emb, W1, b1, W2, b2):
    ids2 = action_ids.T
    flat = ids2.reshape(FIELDS, 2, HALF_B).transpose(0, 2, 1).reshape(-1)
    idx = flat.reshape(NW, NCH, CH)
    staging = _sc_gather(idx, W_emb)
    staging128 = staging.reshape(N_ROWS // 2, 2 * D)
    w1t = W1.T
    zeros = jnp.zeros((D, D), jnp.float32)
    w1l = jnp.concatenate([w1t, zeros], axis=0)
    w1r = jnp.concatenate([zeros, w1t], axis=0)
    y = _mlp(staging128, w1l, w1r, b1.reshape(1, D), W2, b2.reshape(D, 1))
    return y.transpose(2, 0, 1)


# speedup vs baseline: 3.0969x; 1.8104x over previous
import functools

import jax
import jax.numpy as jnp
from jax import lax
from jax.experimental import pallas as pl
from jax.experimental.pallas import tpu as pltpu
from jax.experimental.pallas import tpu_sc as plsc

BATCH = 16384
FIELDS = 26
D = 64
N_ROWS = BATCH * FIELDS
HALF_B = BATCH // 2
V = 1000000

SB = 8192
NSB_MAIN = V // SB
V_MAIN = NSB_MAIN * SB
TAIL = V - V_MAIN
PACK_ROWS = SB // 2
TAIL_PAIRS = TAIL // 2


def _pack_body(x_ref, tail_ref, o_ref):
    k = pl.program_id(0)

    @pl.when(k < NSB_MAIN)
    def _():
        x = x_ref[...]
        o_ref[...] = jnp.concatenate(
            [x[:, :PACK_ROWS].T, x[:, PACK_ROWS:].T], axis=1
        )

    @pl.when(k == NSB_MAIN)
    def _():
        t = tail_ref[...]
        o_ref[pl.ds(0, TAIL_PAIRS), :] = jnp.concatenate(
            [t[:, :TAIL_PAIRS].T, t[:, TAIL_PAIRS:].T], axis=1
        )


def _pack_table(w_emb_t, tail):
    return pl.pallas_call(
        _pack_body,
        grid=(NSB_MAIN + 1,),
        in_specs=[
            pl.BlockSpec((D, SB), lambda k: (0, jnp.minimum(k, NSB_MAIN - 1))),
            pl.BlockSpec((D, TAIL), lambda k: (0, 0)),
        ],
        out_specs=pl.BlockSpec((PACK_ROWS, 2 * D), lambda k: (k, 0)),
        out_shape=jax.ShapeDtypeStruct((V // 2, 2 * D), jnp.float32),
    )(w_emb_t, tail)


NC = 2
NS = 16
NW = NC * NS
BLOCK_PAIRS = 512
BLOCK_ROWS = 2 * BLOCK_PAIRS
NBLK = N_ROWS // BLOCK_ROWS
BLK_PER_W = NBLK // NW
CH = 128
NCH = BLOCK_ROWS // CH
IDS_R = BATCH // CH

_sc_mesh = plsc.VectorSubcoreMesh(
    core_axis_name="c", subcore_axis_name="s", num_cores=NC, num_subcores=NS
)


@functools.partial(
    pl.kernel,
    mesh=_sc_mesh,
    out_type=jax.ShapeDtypeStruct((N_ROWS, D), jnp.float32),
    scratch_types=[
        pltpu.VMEM((4, CH), jnp.int32),
        pltpu.VMEM((4, CH), jnp.int32),
        pltpu.VMEM((BLOCK_ROWS,), jnp.int32),
        pltpu.VMEM((BLOCK_ROWS, D), jnp.float32),
        pltpu.SemaphoreType.DMA,
    ],
    compiler_params=pltpu.CompilerParams(
        use_tc_tiling_on_sc=False, needs_layout_passes=False
    ),
)
def _sc_gather(ids_hbm, table_hbm, out_hbm, lo_v, hi_v, ilv_v, rows_v, gsem):
    wid = lax.axis_index("s") * NC + lax.axis_index("c")
    iota = lax.iota(jnp.int32, 16)

    def block_body(k, carry):
        blk = wid + NW * k
        f = blk // 16
        row0 = (blk % 16) * 4
        pltpu.sync_copy(ids_hbm.at[f, pl.ds(row0, 4)], lo_v)
        pltpu.sync_copy(ids_hbm.at[f, pl.ds(IDS_R // 2 + row0, 4)], hi_v)
        for t in range(BLOCK_PAIRS // 16):
            pos = 2 * (16 * t + iota)
            plsc.store_scatter(ilv_v, [pos], lo_v[t // 8, pl.ds((t % 8) * 16, 16)])
            plsc.store_scatter(ilv_v, [pos + 1], hi_v[t // 8, pl.ds((t % 8) * 16, 16)])
        copies = [
            pltpu.async_copy(
                table_hbm.at[ilv_v.at[pl.ds(j * CH, CH)]],
                rows_v.at[pl.ds(j * CH, CH)],
                gsem,
            )
            for j in range(NCH)
        ]
        for c in copies:
            c.wait()
        pltpu.sync_copy(rows_v, out_hbm.at[pl.ds(blk * BLOCK_ROWS, BLOCK_ROWS)])
        return carry

    lax.fori_loop(0, BLK_PER_W, block_body, 0)


_SQRT_HALF = 0.7071067811865476


def _mlp_body(x_ref, w1l_ref, w1r_ref, b1_ref, w2_ref, b2c_ref, o_ref):
    x = x_ref[...]
    b1 = b1_ref[...]
    b2c = b2c_ref[...]
    w2 = w2_ref[...]
    for half, w1_ref in ((0, w1l_ref), (1, w1r_ref)):
        h = jnp.dot(x, w1_ref[...], preferred_element_type=jnp.float32) + b1
        h = h * 0.5 * (1.0 + lax.erf(h * _SQRT_HALF))
        y = lax.dot_general(
            w2, h, (((1,), (1,)), ((), ())), preferred_element_type=jnp.float32
        )
        o_ref[0, :, pl.ds(half * HALF_B, HALF_B)] = y + b2c


def _mlp(staging128, w1l, w1r, b1, w2, b2c):
    return pl.pallas_call(
        _mlp_body,
        grid=(FIELDS,),
        in_specs=[
            pl.BlockSpec((HALF_B, 2 * D), lambda f: (f, 0)),
            pl.BlockSpec((2 * D, D), lambda f: (0, 0)),
            pl.BlockSpec((2 * D, D), lambda f: (0, 0)),
            pl.BlockSpec((1, D), lambda f: (0, 0)),
            pl.BlockSpec((D, D), lambda f: (0, 0)),
            pl.BlockSpec((D, 1), lambda f: (0, 0)),
        ],
        out_specs=pl.BlockSpec((1, D, BATCH), lambda f: (f, 0, 0)),
        out_shape=jax.ShapeDtypeStruct((FIELDS, D, BATCH), jnp.float32),
    )(staging128, w1l, w1r, b1, w2, b2c)


def kernel(action_ids, W_emb, W1, b1, W2, b2):
    wt = W_emb.T
    tlin = _pack_table(wt, wt[:, V_MAIN:]).reshape(V, D)
    j = action_ids.T
    o = j % SB
    jj_main = 2 * (PACK_ROWS * (j // SB) + o % PACK_ROWS) + o // PACK_ROWS
    ot = j - V_MAIN
    jj_tail = 2 * (V_MAIN // 2 + ot % TAIL_PAIRS) + ot // TAIL_PAIRS
    jj = jnp.where(j < V_MAIN, jj_main, jj_tail)
    ids3 = jj.reshape(FIELDS, IDS_R, CH)
    staging = _sc_gather(ids3, tlin)
    staging128 = staging.reshape(N_ROWS // 2, 2 * D)
    w1t = W1.T
    zeros = jnp.zeros((D, D), jnp.float32)
    w1l = jnp.concatenate([w1t, zeros], axis=0)
    w1r = jnp.concatenate([zeros, w1t], axis=0)
    y = _mlp(staging128, w1l, w1r, b1.reshape(1, D), W2, b2.reshape(D, 1))
    return y.transpose(2, 0, 1)
